# baseline (speedup 1.0000x reference)
import functools

import jax
import jax.numpy as jnp
from jax import lax
from jax.experimental import pallas as pl
from jax.experimental.pallas import tpu as pltpu
from jax.experimental.pallas import tpu_sc as plsc

V = 1000000
D = 64
B = 4096
L = 20
K = 5

NC = 2
NS = 16
NW = NC * NS
BW = B // NW
SCH = 32
NCHUNK = BW // SCH
NG = BW // 16


def _sc_scores_body(tgt_hbm, ctx_hbm, neg_hbm, in_emb, out_emb,
                    pos_out, neg_out,
                    tgt_idx, ctx_idx, neg_idx,
                    t_rows, ctx_rows, neg_rows,
                    pos_buf, neg_buf, sem):
    c_id = lax.axis_index("c")
    s_id = lax.axis_index("s")
    wid = s_id * NC + c_id
    base = wid * BW

    pltpu.sync_copy(tgt_hbm.at[pl.ds(base, BW)], tgt_idx)
    pltpu.sync_copy(ctx_hbm.at[pl.ds(base * L, BW * L)], ctx_idx)
    pltpu.sync_copy(neg_hbm.at[pl.ds(base * K, BW * K)], neg_idx)

    pltpu.async_copy(in_emb.at[tgt_idx], t_rows, sem).wait()

    lanes = lax.iota(jnp.int32, 16)

    for chunk in range(NCHUNK):
        handles = []
        for c in range(5):
            handles.append(pltpu.async_copy(
                out_emb.at[ctx_idx.at[pl.ds(chunk * SCH * L + c * 128, 128)]],
                ctx_rows.at[pl.ds(c * 128, 128)], sem))
        for c in range(2):
            handles.append(pltpu.async_copy(
                out_emb.at[neg_idx.at[pl.ds(chunk * SCH * K + c * 80, 80)]],
                neg_rows.at[pl.ds(c * 80, 80)], sem))
        for h in handles:
            h.wait()

        for g2 in range(2):
            g = chunk * 2 + g2
            t_row = lanes + (chunk * SCH + g2 * 16)
            c_row0 = (lanes + g2 * 16) * L
            n_row0 = (lanes + g2 * 16) * K

            def body(d, accs, t_row=t_row, c_row0=c_row0, n_row0=n_row0):
                dv = jnp.broadcast_to(d, (16,))
                t_col = plsc.load_gather(t_rows, [t_row, dv])
                new = []
                for j in range(L):
                    cc = plsc.load_gather(ctx_rows, [c_row0 + j, dv])
                    new.append(accs[j] + t_col * cc)
                for j in range(K):
                    nc = plsc.load_gather(neg_rows, [n_row0 + j, dv])
                    new.append(accs[L + j] + t_col * nc)
                return tuple(new)

            accs = lax.fori_loop(
                0, D, body,
                tuple(jnp.zeros((16,), jnp.float32) for _ in range(L + K)))

            for j in range(L):
                pos_buf[pl.ds((g * L + j) * 16, 16)] = accs[j]
            for j in range(K):
                neg_buf[pl.ds((g * K + j) * 16, 16)] = accs[L + j]

    pltpu.sync_copy(pos_buf, pos_out.at[wid])
    pltpu.sync_copy(neg_buf, neg_out.at[wid])


_sc_scores = pl.kernel(
    _sc_scores_body,
    out_type=(
        jax.ShapeDtypeStruct((NW, NG * L * 16), jnp.float32),
        jax.ShapeDtypeStruct((NW, NG * K * 16), jnp.float32),
    ),
    mesh=plsc.VectorSubcoreMesh(
        core_axis_name="c", subcore_axis_name="s",
        num_cores=NC, num_subcores=NS),
    compiler_params=pltpu.CompilerParams(needs_layout_passes=False, use_tc_tiling_on_sc=False),
    scratch_types=(
        pltpu.VMEM((BW,), jnp.int32),
        pltpu.VMEM((BW * L,), jnp.int32),
        pltpu.VMEM((BW * K,), jnp.int32),
        pltpu.VMEM((BW, D), jnp.float32),
        pltpu.VMEM((SCH * L, D), jnp.float32),
        pltpu.VMEM((SCH * K, D), jnp.float32),
        pltpu.VMEM((NG * L * 16,), jnp.float32),
        pltpu.VMEM((NG * K * 16,), jnp.float32),
        pltpu.SemaphoreType.DMA,
    ),
)


def _loss_body(pos_ref, neg_ref, out_ref):
    p = pos_ref[...]
    n = neg_ref[...]
    pos_sum = jnp.sum(jnp.maximum(-p, 0.0) + jnp.log1p(jnp.exp(-jnp.abs(p))))
    neg_sum = jnp.sum(jnp.maximum(n, 0.0) + jnp.log1p(jnp.exp(-jnp.abs(n))))
    out_ref[0, 0] = pos_sum / (B * L)
    out_ref[0, 1] = neg_sum / (B * K)


def _loss(pos_scores, neg_scores):
    return pl.pallas_call(
        _loss_body,
        out_shape=jax.ShapeDtypeStruct((1, 2), jnp.float32),
        out_specs=pl.BlockSpec(memory_space=pltpu.SMEM),
    )(pos_scores, neg_scores)


@jax.jit
def kernel(target_words, context_words_list, negative_words, input_emb, output_emb):
    tw = target_words.reshape(-1).astype(jnp.int32)
    cw = context_words_list.reshape(-1).astype(jnp.int32)
    nw = negative_words.reshape(-1).astype(jnp.int32)
    pos_scores, neg_scores = _sc_scores(tw, cw, nw, input_emb, output_emb)
    out = _loss(pos_scores.reshape(B * L // 128, 128),
                neg_scores.reshape(B * K // 128, 128))
    return (out[0, 0], out[0, 1])
